# Initial kernel scaffold; baseline (speedup 1.0000x reference)
import jax
import jax.numpy as jnp
from jax.experimental import pallas as pl


def kernel(data, labels, steps):
    raise NotImplementedError("write your pallas kernel here")


# speedup vs baseline: 11.6182x; 11.6182x over previous
import functools

import jax
import jax.numpy as jnp
import numpy as np
from jax import lax
from jax.experimental import pallas as pl
from jax.experimental.pallas import tpu as pltpu
from jax.experimental.pallas import tpu_sc as plsc

_TW = 10
_TRES = 100
_NX = 64
_NY = 64
_B = 16
_N = _NX * _NY
_E = 94500
_CHUNK0 = 3576
_CHUNKM = 6024
_CHUNKL = 6588
_BUF = 6592


def _row_pattern(trep):
    di_lo, di_hi = max(-2, -trep), min(2, 63 - trep)
    s, d = [], []
    for j in range(64):
        for di in range(di_lo, di_hi + 1):
            for dj in range(max(-2, -j), min(2, 63 - j) + 1):
                if di == 0 and dj == 0:
                    continue
                s.append(j)
                d.append(di * 64 + j + dj)
    return np.asarray(s, np.int32), np.asarray(d, np.int32)


def _chunked(a):
    n = len(a)
    nch = -(-n // 16)
    return (np.concatenate([a[min(16 * u, n - 16):min(16 * u, n - 16) + 16]
                            for u in range(nch)]), n, nch)


def _build_tables():
    parts, meta, off = [], {}, 0
    for trep in (0, 1, 2, 62, 63):
        s, d = _row_pattern(trep)
        cs, n, nch = _chunked(s)
        cd, _, _ = _chunked(d)
        meta[trep] = (off, off + nch * 16, n, nch)
        parts.extend([cs, cd])
        off += 2 * nch * 16
    return np.concatenate(parts), meta


_TBL_NP, _TBL_META = _build_tables()
_TBL_LEN = len(_TBL_NP)


@functools.lru_cache(maxsize=None)
def _edge_sc_fn():
    mesh = plsc.VectorSubcoreMesh(core_axis_name="c", subcore_axis_name="s")

    @functools.partial(
        pl.kernel,
        out_type=[jax.ShapeDtypeStruct((_E,), jnp.int32),
                  jax.ShapeDtypeStruct((_E,), jnp.int32)],
        mesh=mesh,
        scratch_types=[pltpu.VMEM((_TBL_LEN,), jnp.int32),
                       pltpu.VMEM((_BUF,), jnp.int32),
                       pltpu.VMEM((_BUF,), jnp.int32)],
    )
    def _edge_sc(tbl_hbm, src_out, dst_out, tbl, src_buf, dst_buf):
        tid = lax.axis_index("s") * 2 + lax.axis_index("c")

        @pl.when(tid < 16)
        def _():
            k = tid
            pltpu.sync_copy(tbl_hbm, tbl)
            i_start = jnp.where(k == 0, 0, 4 * k - 1)
            i_end = jnp.where(k == 0, 3, jnp.where(k == 15, 64, 4 * k + 3))
            tile_base = (314 * (5 * i_start - jnp.minimum(i_start, 2)
                               - jnp.minimum(i_start, 1)) - 64 * i_start)

            def emit_row(rb, base_i, soff, doff, n, nch):
                def chunk(u, carry):
                    sv = tbl[pl.ds(soff + u * 16, 16)] + base_i
                    src_buf[pl.ds(rb + u * 16, 16)] = sv
                    dv = tbl[pl.ds(doff + u * 16, 16)] + base_i
                    dst_buf[pl.ds(rb + u * 16, 16)] = dv
                    return carry

                lax.fori_loop(0, nch - 1, chunk, jnp.int32(0))
                last = nch - 1
                sv = tbl[pl.ds(soff + last * 16, 16)] + base_i
                src_buf[pl.ds(rb + n - 16, 16)] = sv
                dv = tbl[pl.ds(doff + last * 16, 16)] + base_i
                dst_buf[pl.ds(rb + n - 16, 16)] = dv

            def vbody(v, carry):
                i = i_start + v

                @pl.when(i < i_end)
                def _():
                    wr = (5 * i - jnp.minimum(i, 2) - jnp.minimum(i, 1)
                          - jnp.maximum(i - 62, 0))
                    rb = 314 * wr - 64 * i - tile_base
                    base_i = i * 64
                    for trep, (soff, doff, n, nch) in _TBL_META.items():
                        if trep == 2:
                            cond = (i >= 2) & (i <= 61)
                        else:
                            cond = i == trep
                        pl.when(cond)(
                            lambda so=soff, do=doff, nn=n, nc=nch:
                            emit_row(rb, base_i, so, do, nn, nc))
                return carry

            lax.fori_loop(0, 5, vbody, jnp.int32(0))

            a_k = 3576 + 6024 * (k - 1)

            @pl.when(k == 0)
            def _():
                pltpu.sync_copy(src_buf.at[pl.ds(0, _CHUNK0)],
                                src_out.at[pl.ds(0, _CHUNK0)])
                pltpu.sync_copy(dst_buf.at[pl.ds(0, _CHUNK0)],
                                dst_out.at[pl.ds(0, _CHUNK0)])

            @pl.when((k >= 1) & (k < 15))
            def _():
                pltpu.sync_copy(src_buf.at[pl.ds(0, _CHUNKM)],
                                src_out.at[pl.ds(a_k, _CHUNKM)])
                pltpu.sync_copy(dst_buf.at[pl.ds(0, _CHUNKM)],
                                dst_out.at[pl.ds(a_k, _CHUNKM)])

            @pl.when(k == 15)
            def _():
                pltpu.sync_copy(src_buf.at[pl.ds(0, _CHUNKL)],
                                src_out.at[pl.ds(87912, _CHUNKL)])
                pltpu.sync_copy(dst_buf.at[pl.ds(0, _CHUNKL)],
                                dst_out.at[pl.ds(87912, _CHUNKL)])

    return _edge_sc


def _bcast_body(s_ref, d_ref, o_ref):
    b = pl.program_id(0)
    off = b * _N
    o_ref[0, 0, 0, :] = s_ref[0, :] + off
    o_ref[1, 0, 0, :] = d_ref[0, :] + off


def _edges_all(esrc1, edst1):
    return pl.pallas_call(
        _bcast_body,
        grid=(_B,),
        in_specs=[pl.BlockSpec((1, _E), lambda b: (0, 0)),
                  pl.BlockSpec((1, _E), lambda b: (0, 0))],
        out_specs=pl.BlockSpec((2, 1, 1, _E), lambda b: (0, b, 0, 0)),
        out_shape=jax.ShapeDtypeStruct((2, _B, 1, _E), jnp.int32),
    )(esrc1, edst1)


def _tr_body(d_ref, l_ref, u_ref, y_ref):
    eye = jnp.eye(16, dtype=jnp.float32)
    z = jnp.zeros((6, _N), jnp.float32)
    dp = jnp.concatenate([d_ref[0], z], 0)
    lp = jnp.concatenate([l_ref[0], z], 0)
    dt = lax.dot_general(dp, eye, (((0,), (0,)), ((), ())),
                         preferred_element_type=jnp.float32)
    lt = lax.dot_general(lp, eye, (((0,), (0,)), ((), ())),
                         preferred_element_type=jnp.float32)
    u_ref[0] = dt[:, :_TW]
    y_ref[0] = lt[:, :_TW]


def _transpose_uy(d3, l3):
    return pl.pallas_call(
        _tr_body,
        grid=(_B,),
        in_specs=[pl.BlockSpec((1, _TW, _N), lambda b: (b, 0, 0)),
                  pl.BlockSpec((1, _TW, _N), lambda b: (b, 0, 0))],
        out_specs=[pl.BlockSpec((1, _N, _TW), lambda b: (b, 0, 0)),
                   pl.BlockSpec((1, _N, _TW), lambda b: (b, 0, 0))],
        out_shape=[jax.ShapeDtypeStruct((_B, _N, _TW), jnp.float32),
                   jax.ShapeDtypeStruct((_B, _N, _TW), jnp.float32)],
    )(d3, l3)


def _posb_body(steps_ref, t_ref, grid_ref, pos_ref, batch_ref):
    b = pl.program_id(0)
    s = steps_ref[b, 0]
    tv = t_ref[s, 0]
    pos_ref[0, :, 0:1] = jnp.full((_N, 1), tv, jnp.float32)
    pos_ref[0, :, 1:3] = grid_ref[...]
    batch_ref[0, 0, :] = jnp.full((_N,), b, jnp.int32)


def _pos_batch(steps2, t2, gridc):
    return pl.pallas_call(
        _posb_body,
        grid=(_B,),
        in_specs=[pl.BlockSpec(memory_space=pltpu.SMEM),
                  pl.BlockSpec(memory_space=pltpu.SMEM),
                  pl.BlockSpec((_N, 2), lambda b: (0, 0))],
        out_specs=[pl.BlockSpec((1, _N, 3), lambda b: (b, 0, 0)),
                   pl.BlockSpec((1, 1, _N), lambda b: (b, 0, 0))],
        out_shape=[jax.ShapeDtypeStruct((_B, _N, 3), jnp.float32),
                   jax.ShapeDtypeStruct((_B, 1, _N), jnp.int32)],
    )(steps2, t2, gridc)


def kernel(data, labels, steps):
    b, tw, nx, ny = data.shape
    d3 = data.reshape(b, tw, _N)
    l3 = labels.reshape(b, tw, _N)

    esrc, edst = _edge_sc_fn()(jnp.asarray(_TBL_NP))
    edges = _edges_all(esrc.reshape(1, _E),
                       edst.reshape(1, _E)).reshape(2, _B * _E)
    u, yv = _transpose_uy(d3, l3)

    t_tab = jnp.linspace(0.0, 1.0, _TRES).astype(jnp.float32).reshape(_TRES, 1)
    xs = jnp.linspace(0.0, 1.0, _NX)
    ys = jnp.linspace(0.0, 1.0, _NY)
    gx, gy = jnp.meshgrid(xs, ys, indexing="ij")
    gridc = jnp.stack((gx, gy), 2).astype(jnp.float32).reshape(_N, 2)
    pos3, batch3 = _pos_batch(steps.reshape(_B, 1), t_tab, gridc)

    return (u.reshape(b * _N, tw), edges, yv.reshape(b * _N, tw),
            pos3.reshape(b * _N, 3), batch3.reshape(b * _N))


# speedup vs baseline: 34.6231x; 2.9801x over previous
import functools

import jax
import jax.numpy as jnp
import numpy as np
from jax import lax
from jax.experimental import pallas as pl
from jax.experimental.pallas import tpu as pltpu
from jax.experimental.pallas import tpu_sc as plsc

_TW = 10
_TRES = 100
_NX = 64
_NY = 64
_B = 16
_N = _NX * _NY
_E = 94500
_CHUNK0 = 3576
_CHUNKM = 6024
_CHUNKL = 6588
_BUF = 6592


def _row_pattern(trep):
    di_lo, di_hi = max(-2, -trep), min(2, 63 - trep)
    s, d = [], []
    for j in range(64):
        for di in range(di_lo, di_hi + 1):
            for dj in range(max(-2, -j), min(2, 63 - j) + 1):
                if di == 0 and dj == 0:
                    continue
                s.append(j)
                d.append(di * 64 + j + dj)
    return np.asarray(s, np.int32), np.asarray(d, np.int32)


def _chunked(a):
    n = len(a)
    nch = -(-n // 16)
    return (np.concatenate([a[min(16 * u, n - 16):min(16 * u, n - 16) + 16]
                            for u in range(nch)]), n, nch)


def _build_tables():
    parts, meta, off = [], {}, 0
    for trep in (0, 1, 2, 62, 63):
        s, d = _row_pattern(trep)
        cs, n, nch = _chunked(s)
        cd, _, _ = _chunked(d)
        meta[trep] = (off, off + nch * 16, n, nch)
        parts.extend([cs, cd])
        off += 2 * nch * 16
    return np.concatenate(parts), meta


_TBL_NP, _TBL_META = _build_tables()
_TBL_LEN = len(_TBL_NP)


@functools.lru_cache(maxsize=None)
def _edge_sc_fn():
    mesh = plsc.VectorSubcoreMesh(core_axis_name="c", subcore_axis_name="s")

    @functools.partial(
        pl.kernel,
        out_type=[jax.ShapeDtypeStruct((_E,), jnp.int32),
                  jax.ShapeDtypeStruct((_E,), jnp.int32)],
        mesh=mesh,
        scratch_types=[pltpu.VMEM((_TBL_LEN,), jnp.int32),
                       pltpu.VMEM((_BUF,), jnp.int32),
                       pltpu.VMEM((_BUF,), jnp.int32)],
    )
    def _edge_sc(tbl_hbm, src_out, dst_out, tbl, src_buf, dst_buf):
        tid = lax.axis_index("s") * 2 + lax.axis_index("c")

        @pl.when(tid < 16)
        def _():
            k = tid
            pltpu.sync_copy(tbl_hbm, tbl)
            i_start = jnp.where(k == 0, 0, 4 * k - 1)
            i_end = jnp.where(k == 0, 3, jnp.where(k == 15, 64, 4 * k + 3))
            tile_base = (314 * (5 * i_start - jnp.minimum(i_start, 2)
                               - jnp.minimum(i_start, 1)) - 64 * i_start)

            def emit_row(rb, base_i, soff, doff, n, nch):
                def chunk(u, carry):
                    sv = tbl[pl.ds(soff + u * 16, 16)] + base_i
                    src_buf[pl.ds(rb + u * 16, 16)] = sv
                    dv = tbl[pl.ds(doff + u * 16, 16)] + base_i
                    dst_buf[pl.ds(rb + u * 16, 16)] = dv
                    return carry

                lax.fori_loop(0, nch - 1, chunk, jnp.int32(0))
                last = nch - 1
                sv = tbl[pl.ds(soff + last * 16, 16)] + base_i
                src_buf[pl.ds(rb + n - 16, 16)] = sv
                dv = tbl[pl.ds(doff + last * 16, 16)] + base_i
                dst_buf[pl.ds(rb + n - 16, 16)] = dv

            def vbody(v, carry):
                i = i_start + v

                @pl.when(i < i_end)
                def _():
                    wr = (5 * i - jnp.minimum(i, 2) - jnp.minimum(i, 1)
                          - jnp.maximum(i - 62, 0))
                    rb = 314 * wr - 64 * i - tile_base
                    base_i = i * 64
                    for trep, (soff, doff, n, nch) in _TBL_META.items():
                        if trep == 2:
                            cond = (i >= 2) & (i <= 61)
                        else:
                            cond = i == trep
                        pl.when(cond)(
                            lambda so=soff, do=doff, nn=n, nc=nch:
                            emit_row(rb, base_i, so, do, nn, nc))
                return carry

            lax.fori_loop(0, 5, vbody, jnp.int32(0))

            a_k = 3576 + 6024 * (k - 1)

            @pl.when(k == 0)
            def _():
                pltpu.sync_copy(src_buf.at[pl.ds(0, _CHUNK0)],
                                src_out.at[pl.ds(0, _CHUNK0)])
                pltpu.sync_copy(dst_buf.at[pl.ds(0, _CHUNK0)],
                                dst_out.at[pl.ds(0, _CHUNK0)])

            @pl.when((k >= 1) & (k < 15))
            def _():
                pltpu.sync_copy(src_buf.at[pl.ds(0, _CHUNKM)],
                                src_out.at[pl.ds(a_k, _CHUNKM)])
                pltpu.sync_copy(dst_buf.at[pl.ds(0, _CHUNKM)],
                                dst_out.at[pl.ds(a_k, _CHUNKM)])

            @pl.when(k == 15)
            def _():
                pltpu.sync_copy(src_buf.at[pl.ds(0, _CHUNKL)],
                                src_out.at[pl.ds(87912, _CHUNKL)])
                pltpu.sync_copy(dst_buf.at[pl.ds(0, _CHUNKL)],
                                dst_out.at[pl.ds(87912, _CHUNKL)])

    return _edge_sc


def _fused_body(steps_ref, t_ref, es_ref, ed_ref, gridT_ref, d_ref, l_ref,
                u_ref, y_ref, edges_ref, pos_ref, batch_ref):
    b = pl.program_id(0)
    u_ref[...] = d_ref[0]
    y_ref[...] = l_ref[0]
    off = b * _N
    edges_ref[0, 0, :] = es_ref[0, :] + off
    edges_ref[0, 1, :] = ed_ref[0, :] + off
    s = steps_ref[b, 0]
    tv = t_ref[s, 0]
    pos_ref[0:1, :] = jnp.full((1, _N), tv, jnp.float32)
    pos_ref[1:3, :] = gridT_ref[...]
    batch_ref[...] = jnp.full((_N,), b, jnp.int32)


def _fused(steps2, t2, es1, ed1, gridT, d3, l3):
    return pl.pallas_call(
        _fused_body,
        grid=(_B,),
        in_specs=[pl.BlockSpec(memory_space=pltpu.SMEM),
                  pl.BlockSpec(memory_space=pltpu.SMEM),
                  pl.BlockSpec((1, _E), lambda b: (0, 0)),
                  pl.BlockSpec((1, _E), lambda b: (0, 0)),
                  pl.BlockSpec((2, _N), lambda b: (0, 0)),
                  pl.BlockSpec((1, _TW, _N), lambda b: (b, 0, 0)),
                  pl.BlockSpec((1, _TW, _N), lambda b: (b, 0, 0))],
        out_specs=[pl.BlockSpec((_TW, _N), lambda b: (0, b)),
                   pl.BlockSpec((_TW, _N), lambda b: (0, b)),
                   pl.BlockSpec((1, 2, _E), lambda b: (b, 0, 0)),
                   pl.BlockSpec((3, _N), lambda b: (0, b)),
                   pl.BlockSpec((_N,), lambda b: (b,))],
        out_shape=[jax.ShapeDtypeStruct((_TW, _B * _N), jnp.float32),
                   jax.ShapeDtypeStruct((_TW, _B * _N), jnp.float32),
                   jax.ShapeDtypeStruct((_B, 2, _E), jnp.int32),
                   jax.ShapeDtypeStruct((3, _B * _N), jnp.float32),
                   jax.ShapeDtypeStruct((_B * _N,), jnp.int32)],
    )(steps2, t2, es1, ed1, gridT, d3, l3)


def kernel(data, labels, steps):
    b, tw, nx, ny = data.shape
    d3 = data.reshape(b, tw, _N)
    l3 = labels.reshape(b, tw, _N)

    esrc, edst = _edge_sc_fn()(jnp.asarray(_TBL_NP))

    t_tab = jnp.linspace(0.0, 1.0, _TRES).astype(jnp.float32).reshape(_TRES, 1)
    xs = jnp.linspace(0.0, 1.0, _NX)
    ys = jnp.linspace(0.0, 1.0, _NY)
    gx, gy = jnp.meshgrid(xs, ys, indexing="ij")
    gridT = jnp.stack((gx.reshape(_N), gy.reshape(_N)), 0).astype(jnp.float32)

    uT, yT, edges3, posT, batch1 = _fused(
        steps.reshape(_B, 1), t_tab, esrc.reshape(1, _E),
        edst.reshape(1, _E), gridT, d3, l3)

    edges = jnp.swapaxes(edges3, 0, 1).reshape(2, _B * _E)
    return (uT.T, edges, yT.T, posT.T, batch1)
